# baseline (speedup 1.0000x reference)
import jax
import jax.numpy as jnp
import numpy as np
from jax import lax
from jax.experimental import pallas as pl
from jax.experimental.pallas import tpu as pltpu

_SCORE_T = 0.05
_NMS_T = 0.6
_MAXDET = 300

_NB = 10
_BNP = 2048
_NPAD = _NB * _BNP
_ROWS = _NPAD // 128
_KSLOT = 384


def _fused_body(cls_ref, cent_ref, b_ref, ob_ref, os_ref, ol_ref,
                ws, labs, out_ref):
    B = cls_ref.shape[0]
    pid = pl.program_id(0)
    rows_blk = _BNP // 128

    @pl.when(pid < _NB)
    def _score_phase():
        cls = cls_ref[...]
        m = jnp.max(cls, axis=-1)
        iota_c = lax.broadcasted_iota(
            jnp.int32, cls.shape, 2).astype(jnp.float32)
        lab = jnp.min(jnp.where(cls == m[:, :, None], iota_c, 1e9), axis=-1)
        c0 = cent_ref[:, :, 0]
        w = jnp.where(m > _SCORE_T, jnp.sqrt(c0 * m), -jnp.inf)
        padw = _BNP - w.shape[1]
        wpad = jnp.concatenate(
            [w, jnp.full((B, padw), -jnp.inf, jnp.float32)], axis=-1)
        lpad = jnp.concatenate(
            [lab, jnp.zeros((B, padw), jnp.float32)], axis=-1)
        ws[:, pl.ds(rows_blk * pid, rows_blk), :] = wpad.reshape(
            B, rows_blk, 128)
        labs[:, pl.ds(rows_blk * pid, rows_blk), :] = lpad.reshape(
            B, rows_blk, 128)

    @pl.when(pid == _NB)
    def _nms_phase():
        _nms_impl(b_ref, ob_ref, os_ref, ol_ref, ws, labs, out_ref)


def _nms_impl(b_ref, ob_ref, os_ref, ol_ref, ws, l_ref, out_ref):
    B = out_ref.shape[0]
    NEG = -jnp.inf
    out_ref[...] = jnp.full(out_ref.shape, -1.0, jnp.float32)

    iota_r = lax.broadcasted_iota(jnp.int32, (B, _ROWS), 1).astype(jnp.float32)
    iota_l = lax.broadcasted_iota(jnp.int32, (B, 128), 1).astype(jnp.float32)
    iota_s = lax.broadcasted_iota(
        jnp.int32, (B, _KSLOT), 1).astype(jnp.float32)
    iota_8 = lax.broadcasted_iota(jnp.int32, (B, 8), 1)
    iota_4 = lax.broadcasted_iota(jnp.int32, (B, 4), 1)

    def rmax(v):
        return jnp.max(v, axis=1, keepdims=True)

    def rmin(v):
        return jnp.min(v, axis=1, keepdims=True)

    def rsum(v):
        return jnp.sum(v, axis=1, keepdims=True)

    rm0 = jnp.max(ws[...], axis=2)
    gm0 = rmax(rm0)
    kept0 = [jnp.zeros((B, _KSLOT), jnp.float32) for _ in range(5)]
    k0 = jnp.zeros((B, 1), jnp.float32)
    state0 = (jnp.int32(0), jnp.int32(0), k0, gm0, rm0) + tuple(kept0)

    def cond(state):
        _, _, k, gm, *_ = state
        alive = (k < float(_MAXDET)) & (gm > NEG)
        return jnp.max(jnp.where(alive, 1.0, 0.0)) > 0.0

    def body(state):
        ks0, ks1, k, gm, rm, kx1, ky1, kx2, ky2, kar = state
        alive = (k < float(_MAXDET)) & (gm > NEG)
        r = rmin(jnp.where(rm == gm, iota_r, 3e5))
        r_s0 = jnp.max(r[0:1, :]).astype(jnp.int32)
        r_s1 = jnp.max(r[1:2, :]).astype(jnp.int32)

        def rows(f):
            return jnp.concatenate([f(0, r_s0), f(1, r_s1)], axis=0)

        wrow = rows(lambda b, r_s: ws[b, pl.ds(r_s, 1), :])
        labr = rows(lambda b, r_s: l_ref[b, pl.ds(r_s, 1), :])

        c = rmin(jnp.where(wrow == gm, iota_l, 3e5))
        oh_l = iota_l == c

        c_s0 = jnp.max(c[0:1, :]).astype(jnp.int32)
        c_s1 = jnp.max(c[1:2, :]).astype(jnp.int32)
        i_s0 = 128 * r_s0 + c_s0 - 48 * lax.shift_right_logical(r_s0, 4)
        i_s1 = 128 * r_s1 + c_s1 - 48 * lax.shift_right_logical(r_s1, 4)
        brows = jnp.concatenate(
            [b_ref[0, pl.ds(i_s0, 1), :], b_ref[1, pl.ds(i_s1, 1), :]],
            axis=0)

        def bcoord(q):
            return rsum(jnp.where(iota_4 == q, brows, 0.0))

        bx1, by1, bx2, by2 = bcoord(0), bcoord(1), bcoord(2), bcoord(3)
        blab = rsum(jnp.where(oh_l, labr, 0.0))
        bar = jnp.maximum(bx2 - bx1, 0.0) * jnp.maximum(by2 - by1, 0.0)

        ix1 = jnp.maximum(kx1, bx1)
        iy1 = jnp.maximum(ky1, by1)
        ix2 = jnp.minimum(kx2, bx2)
        iy2 = jnp.minimum(ky2, by2)
        inter = jnp.maximum(ix2 - ix1, 0.0) * jnp.maximum(iy2 - iy1, 0.0)
        union = kar + bar - inter
        iou = jnp.where(union > 0.0, inter / union, 0.0)
        hit = (iou > _NMS_T) & (iota_s < k)
        suppressed = rmax(jnp.where(hit, 1.0, 0.0)) > 0.0
        keep = alive & jnp.logical_not(suppressed)

        row = jnp.where(iota_8 == 0, bx1,
              jnp.where(iota_8 == 1, by1,
              jnp.where(iota_8 == 2, bx2,
              jnp.where(iota_8 == 3, by2,
              jnp.where(iota_8 == 4, gm,
              jnp.where(iota_8 == 5, blab, 0.0))))))
        row = jnp.where(keep, row, -1.0)
        out_ref[0, pl.ds(ks0, 1), :] = row[0:1, :]
        out_ref[1, pl.ds(ks1, 1), :] = row[1:2, :]

        oh_s = (iota_s == k) & keep
        kx1 = jnp.where(oh_s, bx1, kx1)
        ky1 = jnp.where(oh_s, by1, ky1)
        kx2 = jnp.where(oh_s, bx2, kx2)
        ky2 = jnp.where(oh_s, by2, ky2)
        kar = jnp.where(oh_s, bar, kar)

        wrow_new = jnp.where(oh_l, NEG, wrow)
        ws[0, pl.ds(r_s0, 1), :] = wrow_new[0:1, :]
        ws[1, pl.ds(r_s1, 1), :] = wrow_new[1:2, :]
        nr = rmax(wrow_new)
        rm = jnp.where(iota_r == r, nr, rm)
        gm_n = rmax(rm)

        keep_f = jnp.where(keep, 1.0, 0.0)
        ks0_n = ks0 + jnp.max(keep_f[0:1, :]).astype(jnp.int32)
        ks1_n = ks1 + jnp.max(keep_f[1:2, :]).astype(jnp.int32)
        return (ks0_n, ks1_n, k + keep_f, gm_n, rm,
                kx1, ky1, kx2, ky2, kar)

    lax.while_loop(cond, lambda s: body(body(body(body(s)))), state0)

    ob_ref[...] = out_ref[:, :_MAXDET, 0:4]
    os_ref[...] = out_ref[:, :_MAXDET, 4]
    ol_ref[...] = out_ref[:, :_MAXDET, 5].astype(jnp.int32)


@jax.jit
def kernel(boxes, classification, centerness):
    B, N, C = classification.shape
    Bn = N // _NB

    last = _NB - 1
    out_boxes, out_scores, out_labels = pl.pallas_call(
        _fused_body,
        grid=(_NB + 1,),
        in_specs=[
            pl.BlockSpec((B, Bn, C), lambda i: (0, jnp.minimum(i, last), 0)),
            pl.BlockSpec((B, Bn, 1), lambda i: (0, jnp.minimum(i, last), 0)),
            pl.BlockSpec((B, N, 4), lambda i: (0, 0, 0)),
        ],
        out_specs=[
            pl.BlockSpec((B, _MAXDET, 4), lambda i: (0, 0, 0)),
            pl.BlockSpec((B, _MAXDET), lambda i: (0, 0)),
            pl.BlockSpec((B, _MAXDET), lambda i: (0, 0)),
        ],
        out_shape=[
            jax.ShapeDtypeStruct((B, _MAXDET, 4), jnp.float32),
            jax.ShapeDtypeStruct((B, _MAXDET), jnp.float32),
            jax.ShapeDtypeStruct((B, _MAXDET), jnp.int32),
        ],
        scratch_shapes=[
            pltpu.VMEM((B, _ROWS, 128), jnp.float32),
            pltpu.VMEM((B, _ROWS, 128), jnp.float32),
            pltpu.VMEM((B, 304, 8), jnp.float32),
        ],
    )(classification, centerness, boxes)

    return out_boxes, out_scores, out_labels


# speedup vs baseline: 1.0228x; 1.0228x over previous
import jax
import jax.numpy as jnp
import numpy as np
from jax import lax
from jax.experimental import pallas as pl
from jax.experimental.pallas import tpu as pltpu

_SCORE_T = 0.05
_NMS_T = 0.6
_MAXDET = 300

_NB = 10
_BNP = 2048
_NPAD = _NB * _BNP
_ROWS = _NPAD // 128
_KSLOT = 384


def _fused_body(cls_ref, cent_ref, b_ref, ob_ref, os_ref, ol_ref,
                ws, labs, out_ref):
    B = cls_ref.shape[0]
    pid = pl.program_id(0)
    rows_blk = _BNP // 128

    @pl.when(pid < _NB)
    def _score_phase():
        cls = cls_ref[...]
        m = jnp.max(cls, axis=-1)
        iota_c = lax.broadcasted_iota(
            jnp.int32, cls.shape, 2).astype(jnp.float32)
        lab = jnp.min(jnp.where(cls == m[:, :, None], iota_c, 1e9), axis=-1)
        c0 = cent_ref[:, :, 0]
        w = jnp.where(m > _SCORE_T, jnp.sqrt(c0 * m), -jnp.inf)
        padw = _BNP - w.shape[1]
        wpad = jnp.concatenate(
            [w, jnp.full((B, padw), -jnp.inf, jnp.float32)], axis=-1)
        lpad = jnp.concatenate(
            [lab, jnp.zeros((B, padw), jnp.float32)], axis=-1)
        ws[:, pl.ds(rows_blk * pid, rows_blk), :] = wpad.reshape(
            B, rows_blk, 128)
        labs[:, pl.ds(rows_blk * pid, rows_blk), :] = lpad.reshape(
            B, rows_blk, 128)

    @pl.when(pid == _NB)
    def _nms_phase():
        _nms_impl(b_ref, ob_ref, os_ref, ol_ref, ws, labs, out_ref)


def _nms_impl(b_ref, ob_ref, os_ref, ol_ref, ws, l_ref, out_ref):
    B = out_ref.shape[0]
    NEG = -jnp.inf
    out_ref[...] = jnp.full(out_ref.shape, -1.0, jnp.float32)

    iota_r = lax.broadcasted_iota(jnp.int32, (B, _ROWS), 1).astype(jnp.float32)
    iota_l = lax.broadcasted_iota(jnp.int32, (B, 128), 1).astype(jnp.float32)
    iota_s = lax.broadcasted_iota(
        jnp.int32, (B, _KSLOT), 1).astype(jnp.float32)
    iota_8 = lax.broadcasted_iota(jnp.int32, (B, 8), 1)
    iota_4 = lax.broadcasted_iota(jnp.int32, (B, 4), 1)

    def rmax(v):
        return jnp.max(v, axis=1, keepdims=True)

    def rmin(v):
        return jnp.min(v, axis=1, keepdims=True)

    def rsum(v):
        return jnp.sum(v, axis=1, keepdims=True)

    rm0 = jnp.max(ws[...], axis=2)
    gm0 = rmax(rm0)
    kept0 = [jnp.zeros((B, _KSLOT), jnp.float32) for _ in range(5)]
    k0 = jnp.zeros((B, 1), jnp.float32)
    state0 = (jnp.int32(0), jnp.int32(0), k0, gm0, rm0) + tuple(kept0)

    def cond(state):
        _, _, k, gm, *_ = state
        alive = (k < float(_MAXDET)) & (gm > NEG)
        return jnp.max(jnp.where(alive, 1.0, 0.0)) > 0.0

    def body(state):
        ks0, ks1, k, gm, rm, kx1, ky1, kx2, ky2, kar = state
        alive = (k < float(_MAXDET)) & (gm > NEG)
        r = rmin(jnp.where(rm == gm, iota_r, 3e5))
        r_s0 = jnp.max(r[0:1, :]).astype(jnp.int32)
        r_s1 = jnp.max(r[1:2, :]).astype(jnp.int32)

        def rows(f):
            return jnp.concatenate([f(0, r_s0), f(1, r_s1)], axis=0)

        wrow = rows(lambda b, r_s: ws[b, pl.ds(r_s, 1), :])
        labr = rows(lambda b, r_s: l_ref[b, pl.ds(r_s, 1), :])

        c = rmin(jnp.where(wrow == gm, iota_l, 3e5))
        oh_l = iota_l == c

        c_s0 = jnp.max(c[0:1, :]).astype(jnp.int32)
        c_s1 = jnp.max(c[1:2, :]).astype(jnp.int32)
        i_s0 = 128 * r_s0 + c_s0 - 48 * lax.shift_right_logical(r_s0, 4)
        i_s1 = 128 * r_s1 + c_s1 - 48 * lax.shift_right_logical(r_s1, 4)
        brows = jnp.concatenate(
            [b_ref[0, pl.ds(i_s0, 1), :], b_ref[1, pl.ds(i_s1, 1), :]],
            axis=0)

        def bcoord(q):
            return rsum(jnp.where(iota_4 == q, brows, 0.0))

        bx1, by1, bx2, by2 = bcoord(0), bcoord(1), bcoord(2), bcoord(3)
        blab = rsum(jnp.where(oh_l, labr, 0.0))
        bar = jnp.maximum(bx2 - bx1, 0.0) * jnp.maximum(by2 - by1, 0.0)

        ix1 = jnp.maximum(kx1, bx1)
        iy1 = jnp.maximum(ky1, by1)
        ix2 = jnp.minimum(kx2, bx2)
        iy2 = jnp.minimum(ky2, by2)
        inter = jnp.maximum(ix2 - ix1, 0.0) * jnp.maximum(iy2 - iy1, 0.0)
        union = kar + bar - inter
        iou = jnp.where(union > 0.0, inter / union, 0.0)
        hit = (iou > _NMS_T) & (iota_s < k)
        suppressed = rmax(jnp.where(hit, 1.0, 0.0)) > 0.0
        keep = alive & jnp.logical_not(suppressed)

        row = jnp.where(iota_8 == 0, bx1,
              jnp.where(iota_8 == 1, by1,
              jnp.where(iota_8 == 2, bx2,
              jnp.where(iota_8 == 3, by2,
              jnp.where(iota_8 == 4, gm,
              jnp.where(iota_8 == 5, blab, 0.0))))))
        row = jnp.where(keep, row, -1.0)
        out_ref[0, pl.ds(ks0, 1), :] = row[0:1, :]
        out_ref[1, pl.ds(ks1, 1), :] = row[1:2, :]

        oh_s = (iota_s == k) & keep
        kx1 = jnp.where(oh_s, bx1, kx1)
        ky1 = jnp.where(oh_s, by1, ky1)
        kx2 = jnp.where(oh_s, bx2, kx2)
        ky2 = jnp.where(oh_s, by2, ky2)
        kar = jnp.where(oh_s, bar, kar)

        wrow_new = jnp.where(oh_l, NEG, wrow)
        ws[0, pl.ds(r_s0, 1), :] = wrow_new[0:1, :]
        ws[1, pl.ds(r_s1, 1), :] = wrow_new[1:2, :]
        nr = rmax(wrow_new)
        rm = jnp.where(iota_r == r, nr, rm)
        gm_n = rmax(rm)

        keep_f = jnp.where(keep, 1.0, 0.0)
        ks0_n = ks0 + jnp.max(keep_f[0:1, :]).astype(jnp.int32)
        ks1_n = ks1 + jnp.max(keep_f[1:2, :]).astype(jnp.int32)
        return (ks0_n, ks1_n, k + keep_f, gm_n, rm,
                kx1, ky1, kx2, ky2, kar)

    b2 = lambda s: body(body(s))
    lax.while_loop(cond, lambda s: b2(b2(b2(b2(s)))), state0)

    ob_ref[...] = out_ref[:, :_MAXDET, 0:4]
    os_ref[...] = out_ref[:, :_MAXDET, 4]
    ol_ref[...] = out_ref[:, :_MAXDET, 5].astype(jnp.int32)


@jax.jit
def kernel(boxes, classification, centerness):
    B, N, C = classification.shape
    Bn = N // _NB

    last = _NB - 1
    out_boxes, out_scores, out_labels = pl.pallas_call(
        _fused_body,
        grid=(_NB + 1,),
        in_specs=[
            pl.BlockSpec((B, Bn, C), lambda i: (0, jnp.minimum(i, last), 0)),
            pl.BlockSpec((B, Bn, 1), lambda i: (0, jnp.minimum(i, last), 0)),
            pl.BlockSpec((B, N, 4), lambda i: (0, 0, 0)),
        ],
        out_specs=[
            pl.BlockSpec((B, _MAXDET, 4), lambda i: (0, 0, 0)),
            pl.BlockSpec((B, _MAXDET), lambda i: (0, 0)),
            pl.BlockSpec((B, _MAXDET), lambda i: (0, 0)),
        ],
        out_shape=[
            jax.ShapeDtypeStruct((B, _MAXDET, 4), jnp.float32),
            jax.ShapeDtypeStruct((B, _MAXDET), jnp.float32),
            jax.ShapeDtypeStruct((B, _MAXDET), jnp.int32),
        ],
        scratch_shapes=[
            pltpu.VMEM((B, _ROWS, 128), jnp.float32),
            pltpu.VMEM((B, _ROWS, 128), jnp.float32),
            pltpu.VMEM((B, 304, 8), jnp.float32),
        ],
    )(classification, centerness, boxes)

    return out_boxes, out_scores, out_labels
